# baseline (device time: 57686 ns/iter reference)
import jax
import jax.numpy as jnp
from jax import lax
from jax.experimental import pallas as pl
from jax.experimental.pallas import tpu as pltpu

N_DEV = 4
B, SQ, DM = 4, 256, 1024
HL, DH = 8, 128
CH = DM // 2
HSQ = SQ // 2
SCALE = 0.08838834764831843
MESH = pl.DeviceIdType.MESH


def kernel(x, Wq, Wo, Wk, Wv):
    def body(
        x_ref, wq_ref, wo_ref, wk_ref, wv_ref, out_ref,
        ob_ref, p_ref, snd_ref, rsv_ref, ownA_ref, ownB_ref, ag1_ref,
        snd2_ref, ag2_ref,
        rs_ssem, rs_rsem, ag1_ssem, ag1_rsem, ag2_ssem, ag2_rsem,
    ):
        my = lax.axis_index("i")
        right = (my + 1) % N_DEV
        left = (my - 1) % N_DEV

        barrier = pltpu.get_barrier_semaphore()
        for nbr in (left, right):
            pl.semaphore_signal(barrier, inc=1, device_id=(nbr,),
                                device_id_type=MESH)
        pl.semaphore_wait(barrier, 2)

        def attn_chunk(bi, slot):
            xb = x_ref[pl.ds(bi, 1), :, :].reshape(SQ, DM)
            qb = jnp.dot(xb, wq_ref[...], preferred_element_type=jnp.float32)
            kb = jnp.dot(xb, wk_ref[...], preferred_element_type=jnp.float32)
            vb = jnp.dot(xb, wv_ref[...], preferred_element_type=jnp.float32)
            for h in range(HL):
                c0 = h * DH
                q = qb[:, c0:c0 + DH]
                k = kb[:, c0:c0 + DH]
                v = vb[:, c0:c0 + DH]
                s = lax.dot_general(
                    q, k, (((1,), (1,)), ((), ())),
                    preferred_element_type=jnp.float32,
                ) * SCALE
                m = jnp.max(s, axis=-1, keepdims=True)
                p = jnp.exp(s - m)
                l = jnp.sum(p, axis=-1, keepdims=True)
                o = jnp.dot(p, v, preferred_element_type=jnp.float32) / l
                ob_ref[slot, :, c0:c0 + DH] = o

        def proj(bi, slot, lo, hi):
            pc = jnp.dot(ob_ref[slot], wo_ref[:, lo:hi],
                         preferred_element_type=jnp.float32)
            p_ref[pl.ds(bi * SQ, SQ), lo:hi] = pc
            return pc

        def p_sub(bi, s, lo, hi):
            return p_ref[pl.ds(bi * SQ + s * HSQ, HSQ), lo:hi]

        def rs_rdma(ring, hp, s, dst):
            return pltpu.make_async_remote_copy(
                src_ref=snd_ref.at[ring, hp, s],
                dst_ref=rsv_ref.at[ring, hp, s],
                send_sem=rs_ssem.at[ring, hp, s],
                recv_sem=rs_rsem.at[ring, hp, s],
                device_id=(dst,), device_id_type=MESH,
            )

        b_p1 = (my + 1) % N_DEV
        b_m1 = (my - 1) % N_DEV
        b_p2 = (my + 2) % N_DEV

        attn_chunk(my, 0)
        pc = proj(my, 0, 0, DM)
        snd_ref[0, 0, 0] = pc[0:HSQ, 0:CH]
        snd_ref[0, 0, 1] = pc[HSQ:SQ, 0:CH]
        snd_ref[1, 0, 0] = pc[0:HSQ, CH:DM]
        snd_ref[1, 0, 1] = pc[HSQ:SQ, CH:DM]
        rs = {}
        for s in range(2):
            rs[(0, 0, s)] = rs_rdma(0, 0, s, right)
            rs[(1, 0, s)] = rs_rdma(1, 0, s, left)
        rs[(0, 0, 0)].start()
        rs[(1, 0, 0)].start()
        rs[(0, 0, 1)].start()
        rs[(1, 0, 1)].start()

        attn_chunk(b_p1, 0)
        proj(b_p1, 0, CH, DM)
        attn_chunk(b_m1, 1)
        proj(b_m1, 1, 0, CH)

        def forward(ring, hp, bi, lo, hi, dst):
            for s in range(2):
                rs[(ring, hp, s)].wait()
                snd_ref[ring, hp + 1, s] = (
                    rsv_ref[ring, hp, s] + p_sub(bi, s, lo, hi))
                r = rs_rdma(ring, hp + 1, s, dst)
                rs[(ring, hp + 1, s)] = r
                r.start()

        forward(0, 0, b_m1, 0, CH, right)
        forward(1, 0, b_p1, CH, DM, left)

        proj(b_p1, 0, 0, CH)
        proj(b_m1, 1, CH, DM)
        attn_chunk(b_p2, 0)
        proj(b_p2, 0, 0, DM)

        forward(0, 1, b_p2, 0, CH, right)
        forward(1, 1, b_p2, CH, DM, left)

        def ag1_rdma(src, d, part, s, dst):
            return pltpu.make_async_remote_copy(
                src_ref=src, dst_ref=ag1_ref.at[d, part, s],
                send_sem=ag1_ssem.at[d, part, s],
                recv_sem=ag1_rsem.at[d, part, s],
                device_id=(dst,), device_id_type=MESH,
            )

        g1 = {}
        for s in range(2):
            rs[(0, 2, s)].wait()
            ownA_ref[s] = rsv_ref[0, 2, s] + p_sub(b_p1, s, 0, CH)
            g1[(0, 0, s)] = ag1_rdma(ownA_ref.at[s], 0, 0, s, right)
            g1[(0, 0, s)].start()
            rs[(1, 2, s)].wait()
            ownB_ref[s] = rsv_ref[1, 2, s] + p_sub(b_m1, s, CH, DM)
            g1[(1, 0, s)] = ag1_rdma(ownB_ref.at[s], 1, 0, s, left)
            g1[(1, 0, s)].start()
        for s in range(2):
            g1[(0, 1, s)] = ag1_rdma(ownB_ref.at[s], 0, 1, s, right)
            g1[(0, 1, s)].start()
            g1[(1, 1, s)] = ag1_rdma(ownA_ref.at[s], 1, 1, s, left)
            g1[(1, 1, s)].start()
        out_ref[pl.ds(b_p1, 1), :, 0:CH] = ownA_ref[...].reshape(1, SQ, CH)
        out_ref[pl.ds(b_m1, 1), :, CH:DM] = ownB_ref[...].reshape(1, SQ, CH)

        g2 = {}
        for s in range(2):
            g1[(0, 0, s)].wait()
            snd2_ref[0, s] = ag1_ref[0, 0, s]
            g2[(0, s)] = pltpu.make_async_remote_copy(
                src_ref=snd2_ref.at[0, s], dst_ref=ag2_ref.at[0, s],
                send_sem=ag2_ssem.at[0, s], recv_sem=ag2_rsem.at[0, s],
                device_id=(right,), device_id_type=MESH,
            )
            g2[(0, s)].start()
            g1[(1, 0, s)].wait()
            snd2_ref[1, s] = ag1_ref[1, 0, s]
            g2[(1, s)] = pltpu.make_async_remote_copy(
                src_ref=snd2_ref.at[1, s], dst_ref=ag2_ref.at[1, s],
                send_sem=ag2_ssem.at[1, s], recv_sem=ag2_rsem.at[1, s],
                device_id=(left,), device_id_type=MESH,
            )
            g2[(1, s)].start()
        out_ref[pl.ds(my, 1), :, 0:CH] = ag1_ref[0, 0].reshape(1, SQ, CH)
        out_ref[pl.ds(my, 1), :, CH:DM] = ag1_ref[1, 0].reshape(1, SQ, CH)
        g1[(0, 1, 0)].wait()
        g1[(0, 1, 1)].wait()
        g1[(1, 1, 0)].wait()
        g1[(1, 1, 1)].wait()
        out_ref[pl.ds(b_p2, 1), :, 0:CH] = ag1_ref[1, 1].reshape(1, SQ, CH)
        out_ref[pl.ds(b_p2, 1), :, CH:DM] = ag1_ref[0, 1].reshape(1, SQ, CH)
        g2[(0, 0)].wait()
        g2[(0, 1)].wait()
        g2[(1, 0)].wait()
        g2[(1, 1)].wait()
        out_ref[pl.ds(b_m1, 1), :, 0:CH] = ag2_ref[0].reshape(1, SQ, CH)
        out_ref[pl.ds(b_p1, 1), :, CH:DM] = ag2_ref[1].reshape(1, SQ, CH)

    return pl.pallas_call(
        body,
        out_shape=jax.ShapeDtypeStruct((B, SQ, DM), jnp.float32),
        in_specs=[pl.BlockSpec(memory_space=pltpu.VMEM)] * 5,
        out_specs=pl.BlockSpec(memory_space=pltpu.VMEM),
        scratch_shapes=[
            pltpu.VMEM((2, SQ, DM), jnp.float32),
            pltpu.VMEM((B * SQ, DM), jnp.float32),
            pltpu.VMEM((2, 3, 2, HSQ, CH), jnp.float32),
            pltpu.VMEM((2, 3, 2, HSQ, CH), jnp.float32),
            pltpu.VMEM((2, HSQ, CH), jnp.float32),
            pltpu.VMEM((2, HSQ, CH), jnp.float32),
            pltpu.VMEM((2, 2, 2, HSQ, CH), jnp.float32),
            pltpu.VMEM((2, 2, HSQ, CH), jnp.float32),
            pltpu.VMEM((2, 2, HSQ, CH), jnp.float32),
            pltpu.SemaphoreType.DMA((2, 3, 2)),
            pltpu.SemaphoreType.DMA((2, 3, 2)),
            pltpu.SemaphoreType.DMA((2, 2, 2)),
            pltpu.SemaphoreType.DMA((2, 2, 2)),
            pltpu.SemaphoreType.DMA((2, 2)),
            pltpu.SemaphoreType.DMA((2, 2)),
        ],
        compiler_params=pltpu.CompilerParams(
            collective_id=0,
            vmem_limit_bytes=100 * 1024 * 1024,
        ),
    )(x, Wq, Wo, Wk, Wv)
